# Initial kernel scaffold; baseline (speedup 1.0000x reference)
import jax
import jax.numpy as jnp
from jax.experimental import pallas as pl


def kernel(x, W, b):
    raise NotImplementedError("write your pallas kernel here")


# speedup vs baseline: 1.9376x; 1.9376x over previous
import functools

import jax
import jax.numpy as jnp
from jax.experimental import pallas as pl

HIDDEN_DIM = 768
EXPERT_NUMBER = 64
TOP_K = 2
BLOCK_N = 1024

_NEG_INF = float("-inf")


def _router_kernel(x_ref, wt_ref, b_ref, logits_ref, weights_ref, idx_ref,
                   mask_ref):
    x = x_ref[...]
    logits = jnp.dot(x, wt_ref[...], preferred_element_type=jnp.float32)
    logits = logits + b_ref[...][None, :]
    logits_ref[...] = logits

    m1 = jnp.max(logits, axis=1)
    i1 = jnp.argmax(logits, axis=1).astype(jnp.int32)
    e_iota = jax.lax.broadcasted_iota(jnp.int32, logits.shape, 1)
    masked = jnp.where(e_iota == i1[:, None], _NEG_INF, logits)
    m2 = jnp.max(masked, axis=1)
    i2 = jnp.argmax(masked, axis=1).astype(jnp.int32)

    w1 = jax.nn.sigmoid(m1 - m2)
    weights_ref[...] = jnp.stack([w1, 1.0 - w1], axis=1)
    idx_ref[...] = jnp.stack([i1, i2], axis=1)

    bn = logits.shape[0]
    mask_iota = jax.lax.broadcasted_iota(jnp.int32, (EXPERT_NUMBER, TOP_K, bn),
                                         0)
    sel = jnp.stack([i1, i2], axis=0)
    mask_ref[...] = (mask_iota == sel[None, :, :]).astype(jnp.int32)


@functools.partial(jax.jit, static_argnames=())
def kernel(x, W, b):
    n_tokens = x.shape[0]
    grid = (n_tokens // BLOCK_N,)
    wt = W.T
    out_types = (
        jax.ShapeDtypeStruct((n_tokens, EXPERT_NUMBER), jnp.float32),
        jax.ShapeDtypeStruct((n_tokens, TOP_K), jnp.float32),
        jax.ShapeDtypeStruct((n_tokens, TOP_K), jnp.int32),
        jax.ShapeDtypeStruct((EXPERT_NUMBER, TOP_K, n_tokens), jnp.int32),
    )
    logits, weights, idx, mask = pl.pallas_call(
        _router_kernel,
        grid=grid,
        in_specs=[
            pl.BlockSpec((BLOCK_N, HIDDEN_DIM), lambda i: (i, 0)),
            pl.BlockSpec((HIDDEN_DIM, EXPERT_NUMBER), lambda i: (0, 0)),
            pl.BlockSpec((EXPERT_NUMBER,), lambda i: (0,)),
        ],
        out_specs=[
            pl.BlockSpec((BLOCK_N, EXPERT_NUMBER), lambda i: (i, 0)),
            pl.BlockSpec((BLOCK_N, TOP_K), lambda i: (i, 0)),
            pl.BlockSpec((BLOCK_N, TOP_K), lambda i: (i, 0)),
            pl.BlockSpec((EXPERT_NUMBER, TOP_K, BLOCK_N), lambda i: (0, 0, i)),
        ],
        out_shape=out_types,
    )(x, wt, b)
    return (logits, weights, idx, mask)


# speedup vs baseline: 2.0814x; 1.0742x over previous
import functools

import jax
import jax.numpy as jnp
from jax.experimental import pallas as pl

HIDDEN_DIM = 768
EXPERT_NUMBER = 64
TOP_K = 2
BLOCK_N = 2048

_NEG_INF = float("-inf")


def _router_kernel(x_ref, wt_ref, b_ref, logits_ref, weights_ref, idx_ref,
                   mask_ref):
    x = x_ref[...]
    logits = jnp.dot(x, wt_ref[...], preferred_element_type=jnp.float32)
    logits = logits + b_ref[...][None, :]
    logits_ref[...] = logits

    m1 = jnp.max(logits, axis=1)
    i1 = jnp.argmax(logits, axis=1).astype(jnp.int32)
    e_iota = jax.lax.broadcasted_iota(jnp.int32, logits.shape, 1)
    masked = jnp.where(e_iota == i1[:, None], _NEG_INF, logits)
    m2 = jnp.max(masked, axis=1)
    i2 = jnp.argmax(masked, axis=1).astype(jnp.int32)

    w1 = jax.nn.sigmoid(m1 - m2)
    weights_ref[...] = jnp.stack([w1, 1.0 - w1], axis=1)
    idx_ref[...] = jnp.stack([i1, i2], axis=1)

    bn = logits.shape[0]
    mask_iota = jax.lax.broadcasted_iota(jnp.int32, (EXPERT_NUMBER, TOP_K, bn),
                                         0)
    sel = jnp.stack([i1, i2], axis=0)
    mask_ref[...] = (mask_iota == sel[None, :, :]).astype(jnp.int32)


@functools.partial(jax.jit, static_argnames=())
def kernel(x, W, b):
    n_tokens = x.shape[0]
    grid = (n_tokens // BLOCK_N,)
    wt = W.T
    out_types = (
        jax.ShapeDtypeStruct((n_tokens, EXPERT_NUMBER), jnp.float32),
        jax.ShapeDtypeStruct((n_tokens, TOP_K), jnp.float32),
        jax.ShapeDtypeStruct((n_tokens, TOP_K), jnp.int32),
        jax.ShapeDtypeStruct((EXPERT_NUMBER, TOP_K, n_tokens), jnp.int32),
    )
    logits, weights, idx, mask = pl.pallas_call(
        _router_kernel,
        grid=grid,
        in_specs=[
            pl.BlockSpec((BLOCK_N, HIDDEN_DIM), lambda i: (i, 0)),
            pl.BlockSpec((HIDDEN_DIM, EXPERT_NUMBER), lambda i: (0, 0)),
            pl.BlockSpec((EXPERT_NUMBER,), lambda i: (0,)),
        ],
        out_specs=[
            pl.BlockSpec((BLOCK_N, EXPERT_NUMBER), lambda i: (i, 0)),
            pl.BlockSpec((BLOCK_N, TOP_K), lambda i: (i, 0)),
            pl.BlockSpec((BLOCK_N, TOP_K), lambda i: (i, 0)),
            pl.BlockSpec((EXPERT_NUMBER, TOP_K, BLOCK_N), lambda i: (0, 0, i)),
        ],
        out_shape=out_types,
    )(x, wt, b)
    return (logits, weights, idx, mask)


# speedup vs baseline: 2.1683x; 1.0417x over previous
import functools

import jax
import jax.numpy as jnp
from jax.experimental import pallas as pl

HIDDEN_DIM = 768
EXPERT_NUMBER = 64
TOP_K = 2
BLOCK_N = 4096

_NEG_INF = float("-inf")


def _router_kernel(x_ref, wt_ref, b_ref, logits_ref, weights_ref, idx_ref,
                   mask_ref):
    x = x_ref[...]
    logits = jnp.dot(x, wt_ref[...], preferred_element_type=jnp.float32)
    logits = logits + b_ref[...][None, :]
    logits_ref[...] = logits

    m1 = jnp.max(logits, axis=1)
    i1 = jnp.argmax(logits, axis=1).astype(jnp.int32)
    e_iota = jax.lax.broadcasted_iota(jnp.int32, logits.shape, 1)
    masked = jnp.where(e_iota == i1[:, None], _NEG_INF, logits)
    m2 = jnp.max(masked, axis=1)
    i2 = jnp.argmax(masked, axis=1).astype(jnp.int32)

    w1 = jax.nn.sigmoid(m1 - m2)
    weights_ref[...] = jnp.stack([w1, 1.0 - w1], axis=1)
    idx_ref[...] = jnp.stack([i1, i2], axis=1)

    bn = logits.shape[0]
    mask_iota = jax.lax.broadcasted_iota(jnp.int32, (EXPERT_NUMBER, TOP_K, bn),
                                         0)
    sel = jnp.stack([i1, i2], axis=0)
    mask_ref[...] = (mask_iota == sel[None, :, :]).astype(jnp.int32)


@functools.partial(jax.jit, static_argnames=())
def kernel(x, W, b):
    n_tokens = x.shape[0]
    grid = (n_tokens // BLOCK_N,)
    wt = W.T
    out_types = (
        jax.ShapeDtypeStruct((n_tokens, EXPERT_NUMBER), jnp.float32),
        jax.ShapeDtypeStruct((n_tokens, TOP_K), jnp.float32),
        jax.ShapeDtypeStruct((n_tokens, TOP_K), jnp.int32),
        jax.ShapeDtypeStruct((EXPERT_NUMBER, TOP_K, n_tokens), jnp.int32),
    )
    logits, weights, idx, mask = pl.pallas_call(
        _router_kernel,
        grid=grid,
        in_specs=[
            pl.BlockSpec((BLOCK_N, HIDDEN_DIM), lambda i: (i, 0)),
            pl.BlockSpec((HIDDEN_DIM, EXPERT_NUMBER), lambda i: (0, 0)),
            pl.BlockSpec((EXPERT_NUMBER,), lambda i: (0,)),
        ],
        out_specs=[
            pl.BlockSpec((BLOCK_N, EXPERT_NUMBER), lambda i: (i, 0)),
            pl.BlockSpec((BLOCK_N, TOP_K), lambda i: (i, 0)),
            pl.BlockSpec((BLOCK_N, TOP_K), lambda i: (i, 0)),
            pl.BlockSpec((EXPERT_NUMBER, TOP_K, BLOCK_N), lambda i: (0, 0, i)),
        ],
        out_shape=out_types,
    )(x, wt, b)
    return (logits, weights, idx, mask)


# speedup vs baseline: 2.1688x; 1.0002x over previous
import functools

import jax
import jax.numpy as jnp
from jax.experimental import pallas as pl
from jax.experimental.pallas import tpu as pltpu

HIDDEN_DIM = 768
EXPERT_NUMBER = 64
TOP_K = 2
BLOCK_N = 4096

_NEG_INF = float("-inf")


def _router_kernel(x_ref, wt_ref, b_ref, logits_ref, weights_ref, idx_ref,
                   mask_ref):
    x = x_ref[...]
    logits = jnp.dot(x, wt_ref[...], preferred_element_type=jnp.float32)
    logits = logits + b_ref[...][None, :]
    logits_ref[...] = logits

    m1 = jnp.max(logits, axis=1)
    i1 = jnp.argmax(logits, axis=1).astype(jnp.int32)
    e_iota = jax.lax.broadcasted_iota(jnp.int32, logits.shape, 1)
    masked = jnp.where(e_iota == i1[:, None], _NEG_INF, logits)
    m2 = jnp.max(masked, axis=1)
    i2 = jnp.argmax(masked, axis=1).astype(jnp.int32)

    w1 = jax.nn.sigmoid(m1 - m2)
    weights_ref[...] = jnp.stack([w1, 1.0 - w1], axis=1)
    idx_ref[...] = jnp.stack([i1, i2], axis=1)

    bn = logits.shape[0]
    mask_iota = jax.lax.broadcasted_iota(jnp.int32, (EXPERT_NUMBER, TOP_K, bn),
                                         0)
    sel = jnp.stack([i1, i2], axis=0)
    mask_ref[...] = (mask_iota == sel[None, :, :]).astype(jnp.int32)


@functools.partial(jax.jit, static_argnames=())
def kernel(x, W, b):
    n_tokens = x.shape[0]
    grid = (n_tokens // BLOCK_N,)
    wt = W.T
    out_types = (
        jax.ShapeDtypeStruct((n_tokens, EXPERT_NUMBER), jnp.float32),
        jax.ShapeDtypeStruct((n_tokens, TOP_K), jnp.float32),
        jax.ShapeDtypeStruct((n_tokens, TOP_K), jnp.int32),
        jax.ShapeDtypeStruct((EXPERT_NUMBER, TOP_K, n_tokens), jnp.int32),
    )
    logits, weights, idx, mask = pl.pallas_call(
        _router_kernel,
        grid=grid,
        in_specs=[
            pl.BlockSpec((BLOCK_N, HIDDEN_DIM), lambda i: (i, 0)),
            pl.BlockSpec((HIDDEN_DIM, EXPERT_NUMBER), lambda i: (0, 0)),
            pl.BlockSpec((EXPERT_NUMBER,), lambda i: (0,)),
        ],
        out_specs=[
            pl.BlockSpec((BLOCK_N, EXPERT_NUMBER), lambda i: (i, 0)),
            pl.BlockSpec((BLOCK_N, TOP_K), lambda i: (i, 0)),
            pl.BlockSpec((BLOCK_N, TOP_K), lambda i: (i, 0)),
            pl.BlockSpec((EXPERT_NUMBER, TOP_K, BLOCK_N), lambda i: (0, 0, i)),
        ],
        out_shape=out_types,
        compiler_params=pltpu.CompilerParams(
            dimension_semantics=("parallel",)),
    )(x, wt, b)
    return (logits, weights, idx, mask)
